# baseline (speedup 1.0000x reference)
import functools

import jax
import jax.numpy as jnp
from jax import lax
from jax.experimental import pallas as pl
from jax.experimental.pallas import tpu as pltpu
from jax.experimental.pallas import tpu_sc as plsc

_MINOR = 128
_LANES = 16
_NW = 32
_CHUNK = 256


def _sc_body(nr_total, nr_pos, x_hbm, pos_hbm, out_hbm, xbuf, pbuf, sem_x, sem_p):
    cid = lax.axis_index("c")
    sid = lax.axis_index("s")
    wid = sid * 2 + cid
    nr_per_w = nr_total // _NW
    base = wid * nr_per_w
    pos_base = lax.rem(base, nr_pos)
    n_chunks = nr_per_w // _CHUNK

    def chunk(k, carry):
        off = k * _CHUNK
        cp_x = pltpu.make_async_copy(
            x_hbm.at[pl.ds(base + off, _CHUNK)], xbuf, sem_x)
        cp_p = pltpu.make_async_copy(
            pos_hbm.at[pl.ds(pos_base + off, _CHUNK)], pbuf, sem_p)
        cp_x.start()
        cp_p.start()
        cp_x.wait()
        cp_p.wait()

        def add_row(i):
            for j in range(_MINOR // _LANES):
                sl = pl.ds(j * _LANES, _LANES)
                plsc.addupdate(xbuf.at[i, sl], pbuf[i, sl])

        plsc.parallel_loop(0, _CHUNK, 1, unroll=4)(add_row)
        pltpu.sync_copy(xbuf, out_hbm.at[pl.ds(base + off, _CHUNK)])
        return carry

    lax.fori_loop(0, n_chunks, chunk, 0)


def kernel(x, pos_table):
    B, L, D = x.shape
    nr_total = B * L * D // _MINOR
    nr_pos = L * D // _MINOR
    x2 = x.reshape(nr_total, _MINOR)
    pos2 = pos_table[:L].reshape(nr_pos, _MINOR)

    mesh = plsc.VectorSubcoreMesh(core_axis_name="c", subcore_axis_name="s")
    sc = pl.kernel(
        functools.partial(_sc_body, nr_total, nr_pos),
        out_type=jax.ShapeDtypeStruct((nr_total, _MINOR), jnp.float32),
        mesh=mesh,
        scratch_types=[
            pltpu.VMEM((_CHUNK, _MINOR), jnp.float32),
            pltpu.VMEM((_CHUNK, _MINOR), jnp.float32),
            pltpu.SemaphoreType.DMA,
            pltpu.SemaphoreType.DMA,
        ],
    )
    out = sc(x2, pos2)
    return out.reshape(B, L, D)


# speedup vs baseline: 1.2216x; 1.2216x over previous
import functools

import jax
import jax.numpy as jnp
from jax import lax
from jax.experimental import pallas as pl
from jax.experimental.pallas import tpu as pltpu
from jax.experimental.pallas import tpu_sc as plsc

_MINOR = 128
_LANES = 16
_NW = 32
_CH = 128


def _sc_body(nr_pos, n_batch, x_hbm, pos_hbm, out_hbm, xbuf, pbuf,
             sx0, sx1, sx2, st0, st1, st2, sp):
    cid = lax.axis_index("c")
    sid = lax.axis_index("s")
    wid = sid * 2 + cid
    pos_per_w = nr_pos // _NW
    n_chunks = pos_per_w // _CH
    pos_lo = wid * pos_per_w

    sem_x = [sx0, sx1, sx2]
    sem_st = [st0, st1, st2]
    steps = [(c, b) for c in range(n_chunks) for b in range(n_batch)]
    n_steps = len(steps)

    def x_row(c, b):
        return b * nr_pos + pos_lo + c * _CH

    def start_load_x(i):
        c, b = steps[i]
        slot = i % 3
        pltpu.make_async_copy(
            x_hbm.at[pl.ds(x_row(c, b), _CH)], xbuf.at[slot], sem_x[slot]
        ).start()

    def start_load_p(c):
        pltpu.make_async_copy(
            pos_hbm.at[pl.ds(pos_lo + c * _CH, _CH)], pbuf.at[c % 2], sp
        ).start()

    start_load_p(0)
    start_load_x(0)
    for i, (c, b) in enumerate(steps):
        slot = i % 3
        if b == 0:
            pltpu.make_async_copy(
                pos_hbm.at[pl.ds(pos_lo + c * _CH, _CH)], pbuf.at[c % 2], sp
            ).wait()
            if c + 1 < n_chunks:
                start_load_p(c + 1)
        if i + 1 < n_steps:
            if i + 1 >= 3:
                j = i - 2
                jc, jb = steps[j]
                pltpu.make_async_copy(
                    xbuf.at[(i + 1) % 3],
                    out_hbm.at[pl.ds(x_row(jc, jb), _CH)],
                    sem_st[(i + 1) % 3],
                ).wait()
            start_load_x(i + 1)
        pltpu.make_async_copy(
            x_hbm.at[pl.ds(x_row(c, b), _CH)], xbuf.at[slot], sem_x[slot]
        ).wait()

        def add_row(r, slot=slot, pslot=c % 2):
            for j in range(_MINOR // _LANES):
                sl = pl.ds(j * _LANES, _LANES)
                plsc.addupdate(xbuf.at[slot, r, sl], pbuf[pslot, r, sl])

        plsc.parallel_loop(0, _CH, 1, unroll=4)(add_row)
        pltpu.make_async_copy(
            xbuf.at[slot], out_hbm.at[pl.ds(x_row(c, b), _CH)], sem_st[slot]
        ).start()

    for i in range(max(0, n_steps - 3), n_steps):
        c, b = steps[i]
        pltpu.make_async_copy(
            xbuf.at[i % 3], out_hbm.at[pl.ds(x_row(c, b), _CH)], sem_st[i % 3]
        ).wait()


def kernel(x, pos_table):
    B, L, D = x.shape
    nr_total = B * L * D // _MINOR
    nr_pos = L * D // _MINOR
    x2 = x.reshape(nr_total, _MINOR)
    pos2 = pos_table[:L].reshape(nr_pos, _MINOR)

    mesh = plsc.VectorSubcoreMesh(core_axis_name="c", subcore_axis_name="s")
    sc = pl.kernel(
        functools.partial(_sc_body, nr_pos, B),
        out_type=jax.ShapeDtypeStruct((nr_total, _MINOR), jnp.float32),
        mesh=mesh,
        scratch_types=[
            pltpu.VMEM((3, _CH, _MINOR), jnp.float32),
            pltpu.VMEM((2, _CH, _MINOR), jnp.float32),
            pltpu.SemaphoreType.DMA,
            pltpu.SemaphoreType.DMA,
            pltpu.SemaphoreType.DMA,
            pltpu.SemaphoreType.DMA,
            pltpu.SemaphoreType.DMA,
            pltpu.SemaphoreType.DMA,
            pltpu.SemaphoreType.DMA,
        ],
    )
    out = sc(x2, pos2)
    return out.reshape(B, L, D)
